# baseline (device time: 241392 ns/iter reference)
import jax
import jax.numpy as jnp
from jax import lax
from jax.experimental import pallas as pl
from jax.experimental.pallas import tpu as pltpu

N_DEV = 32
N_ROWS = 1024
D_MODEL = 256
D_FF = 512
N_EXPERTS = 128
E_PER_DEV = N_EXPERTS // N_DEV
CHUNK = N_ROWS // N_DEV


def kernel(x, router_W, route_idx, expert_W, shared_W):
    def body(x_ref, rw_ref, idx_ref, ew_ref, sw_ref, out_ref,
             acc_ref, comm_ref, send_sems, recv_sems, credit_sem):
        me = lax.axis_index("i")
        left = (me - 1) % N_DEV
        right = (me + 1) % N_DEV

        barrier = pltpu.get_barrier_semaphore()
        for nbr in (left, right):
            pl.semaphore_signal(barrier, inc=1, device_id=(nbr,),
                                device_id_type=pl.DeviceIdType.MESH)
        pl.semaphore_wait(barrier, 2)

        xf = x_ref[:, :]
        xb = xf.astype(jnp.bfloat16)
        scores = jnp.dot(xf, rw_ref[:, :], preferred_element_type=jnp.float32)
        smax = jnp.max(scores, axis=-1, keepdims=True)
        p = jnp.exp(scores - smax)
        probs = p / jnp.sum(p, axis=-1, keepdims=True)
        idx = idx_ref[:, :]
        cols = lax.broadcasted_iota(jnp.int32, (N_ROWS, N_EXPERTS), 1)
        w = jnp.sum(jnp.where(cols == idx, probs, 0.0), axis=-1, keepdims=True)

        partial = jnp.zeros((N_ROWS, D_FF), jnp.float32)
        for j in range(E_PER_DEV):
            e = me * E_PER_DEV + j
            y = jnp.dot(xb, ew_ref[j].astype(jnp.bfloat16),
                        preferred_element_type=jnp.float32)
            gate = jnp.where(idx == e, w, 0.0)
            partial = partial + gate * y

        acc_ref[:, :] = partial.astype(jnp.bfloat16)
        out_ref[:, :] = jnp.dot(xb, sw_ref[:, :].astype(jnp.bfloat16),
                                preferred_element_type=jnp.float32)

        for s in range(2 * (N_DEV - 1)):
            slot = s % 2
            if s >= 2:
                pl.semaphore_wait(credit_sem, 1)
            in_rs = s < N_DEV - 1
            if in_rs:
                send_chunk = (me - s) % N_DEV
            else:
                t = s - (N_DEV - 1)
                send_chunk = (me + 1 - t) % N_DEV
            src = acc_ref.at[pl.ds(send_chunk * CHUNK, CHUNK)]
            if in_rs:
                dst = comm_ref.at[slot]
            else:
                dst = acc_ref.at[pl.ds(send_chunk * CHUNK, CHUNK)]
            rdma = pltpu.make_async_remote_copy(
                src_ref=src, dst_ref=dst,
                send_sem=send_sems.at[slot], recv_sem=recv_sems.at[slot],
                device_id=(right,), device_id_type=pl.DeviceIdType.MESH)
            rdma.start()
            rdma.wait()
            if in_rs:
                rc = (me - s - 1) % N_DEV
                acc_ref[pl.ds(rc * CHUNK, CHUNK)] = (
                    acc_ref[pl.ds(rc * CHUNK, CHUNK)] + comm_ref[slot])
            pl.semaphore_signal(credit_sem, inc=1, device_id=(left,),
                                device_id_type=pl.DeviceIdType.MESH)
        pl.semaphore_wait(credit_sem, 2)

        out_ref[:, :] = out_ref[:, :] + acc_ref[:, :].astype(jnp.float32)

    return pl.pallas_call(
        body,
        out_shape=jax.ShapeDtypeStruct((N_ROWS, D_FF), jnp.float32),
        in_specs=[pl.BlockSpec(memory_space=pltpu.VMEM)] * 5,
        out_specs=pl.BlockSpec(memory_space=pltpu.VMEM),
        scratch_shapes=[
            pltpu.VMEM((N_ROWS, D_FF), jnp.bfloat16),
            pltpu.VMEM((2, CHUNK, D_FF), jnp.bfloat16),
            pltpu.SemaphoreType.DMA((2,)),
            pltpu.SemaphoreType.DMA((2,)),
            pltpu.SemaphoreType.REGULAR,
        ],
        compiler_params=pltpu.CompilerParams(collective_id=0),
    )(x, router_W, route_idx, expert_W, shared_W)


# device time: 58929 ns/iter; 4.0963x vs baseline; 4.0963x over previous
import jax
import jax.numpy as jnp
from jax import lax
from jax.experimental import pallas as pl
from jax.experimental.pallas import tpu as pltpu

N_DEV = 32
N_ROWS = 1024
D_MODEL = 256
D_FF = 512
N_EXPERTS = 128
E_PER_DEV = N_EXPERTS // N_DEV


def kernel(x, router_W, route_idx, expert_W, shared_W):
    def body(x_ref, rw_ref, idx_ref, ew_ref, sw_ref, out_ref,
             acc_ref, comm_ref, send_sems, recv_sems, exit_sem):
        me = lax.axis_index("i")
        cz = me // 8
        q = me % 8
        cy = q // 2
        cx = (q % 2) ^ (cy % 2)

        def pos(px, py, pz):
            return pz * 8 + py * 2 + (px ^ (py % 2))

        steps = [
            (pos(1 - cx, cy, cz), cx, 512),
            (pos(cx, cy ^ 2, cz), (cy // 2) % 2, 256),
            (pos(cx, cy ^ 1, cz), cy % 2, 128),
            (pos(cx, cy, cz ^ 2), (cz // 2) % 2, 64),
            (pos(cx, cy, cz ^ 1), cz % 2, 32),
        ]

        barrier = pltpu.get_barrier_semaphore()
        for p, _, _ in steps:
            pl.semaphore_signal(barrier, inc=1, device_id=(p,),
                                device_id_type=pl.DeviceIdType.MESH)
        pl.semaphore_wait(barrier, len(steps))

        xf = x_ref[:, :]
        xb = xf.astype(jnp.bfloat16)
        scores = jnp.dot(xf, rw_ref[:, :], preferred_element_type=jnp.float32)
        smax = jnp.max(scores, axis=-1, keepdims=True)
        p_ = jnp.exp(scores - smax)
        probs = p_ / jnp.sum(p_, axis=-1, keepdims=True)
        idx = idx_ref[:, :]
        cols = lax.broadcasted_iota(jnp.int32, (N_ROWS, N_EXPERTS), 1)
        w = jnp.sum(jnp.where(cols == idx, probs, 0.0), axis=-1, keepdims=True)

        partial = jnp.zeros((N_ROWS, D_FF), jnp.float32)
        for j in range(E_PER_DEV):
            e = me * E_PER_DEV + j
            yj = jnp.dot(xb, ew_ref[j].astype(jnp.bfloat16),
                         preferred_element_type=jnp.float32)
            gate = jnp.where(idx == e, w, 0.0)
            partial = partial + gate * yj

        acc_ref[:, :] = partial.astype(jnp.bfloat16)
        out_ref[:, :] = jnp.dot(xb, sw_ref[:, :].astype(jnp.bfloat16),
                                preferred_element_type=jnp.float32)

        off = 0
        stage = 0
        for k, (partner, bit, sz) in enumerate(steps):
            my_off = off + bit * sz
            peer_off = off + (1 - bit) * sz
            rdma = pltpu.make_async_remote_copy(
                src_ref=acc_ref.at[pl.ds(peer_off, sz)],
                dst_ref=comm_ref.at[pl.ds(stage, sz)],
                send_sem=send_sems.at[k], recv_sem=recv_sems.at[k],
                device_id=(partner,), device_id_type=pl.DeviceIdType.MESH)
            rdma.start()
            rdma.wait()
            acc_ref[pl.ds(my_off, sz)] = (
                acc_ref[pl.ds(my_off, sz)] + comm_ref[pl.ds(stage, sz)])
            off = my_off
            stage += sz

        for k in range(len(steps)):
            partner, bit, sz = steps[len(steps) - 1 - k]
            rdma = pltpu.make_async_remote_copy(
                src_ref=acc_ref.at[pl.ds(off, sz)],
                dst_ref=acc_ref.at[pl.ds(off, sz)],
                send_sem=send_sems.at[len(steps) + k],
                recv_sem=recv_sems.at[len(steps) + k],
                device_id=(partner,), device_id_type=pl.DeviceIdType.MESH)
            rdma.start()
            rdma.wait()
            off = off - bit * sz

        out_ref[:, :] = out_ref[:, :] + acc_ref[:, :].astype(jnp.float32)

        for p, _, _ in steps:
            pl.semaphore_signal(exit_sem, inc=1, device_id=(p,),
                                device_id_type=pl.DeviceIdType.MESH)
        pl.semaphore_wait(exit_sem, len(steps))

    return pl.pallas_call(
        body,
        out_shape=jax.ShapeDtypeStruct((N_ROWS, D_FF), jnp.float32),
        in_specs=[pl.BlockSpec(memory_space=pltpu.VMEM)] * 5,
        out_specs=pl.BlockSpec(memory_space=pltpu.VMEM),
        scratch_shapes=[
            pltpu.VMEM((N_ROWS, D_FF), jnp.bfloat16),
            pltpu.VMEM((992, D_FF), jnp.bfloat16),
            pltpu.SemaphoreType.DMA((10,)),
            pltpu.SemaphoreType.DMA((10,)),
            pltpu.SemaphoreType.REGULAR,
        ],
        compiler_params=pltpu.CompilerParams(collective_id=0),
    )(x, router_W, route_idx, expert_W, shared_W)


# device time: 54264 ns/iter; 4.4485x vs baseline; 1.0860x over previous
import jax
import jax.numpy as jnp
from jax import lax
from jax.experimental import pallas as pl
from jax.experimental.pallas import tpu as pltpu

N_DEV = 32
N_ROWS = 1024
HALF = N_ROWS // 2
D_MODEL = 256
D_FF = 512
N_EXPERTS = 128
E_PER_DEV = N_EXPERTS // N_DEV


def kernel(x, router_W, route_idx, expert_W, shared_W):
    def body(x_ref, rw_ref, idx_ref, ew_ref, sw_ref, out_ref,
             acc_ref, comm_ref, send_sems, recv_sems, exit_sem):
        me = lax.axis_index("i")
        cz = me // 8
        q = me % 8
        cy = q // 2
        cx = (q % 2) ^ (cy % 2)

        def pos(px, py, pz):
            return pz * 8 + py * 2 + (px ^ (py % 2))

        steps = [
            (pos(1 - cx, cy, cz), cx, 512),
            (pos(cx, cy ^ 1, cz), cy % 2, 256),
            (pos(cx, cy ^ 2, cz), (cy // 2) % 2, 128),
            (pos(cx, cy, cz ^ 1), cz % 2, 64),
            (pos(cx, cy, cz ^ 2), (cz // 2) % 2, 32),
        ]

        barrier = pltpu.get_barrier_semaphore()
        for p, _, _ in steps:
            pl.semaphore_signal(barrier, inc=1, device_id=(p,),
                                device_id_type=pl.DeviceIdType.MESH)
        pl.semaphore_wait(barrier, len(steps))

        def compute_partial(row_off):
            xh = x_ref[pl.ds(row_off, HALF), :]
            xbh = xh.astype(jnp.bfloat16)
            scores = jnp.dot(xh, rw_ref[:, :],
                             preferred_element_type=jnp.float32)
            smax = jnp.max(scores, axis=-1, keepdims=True)
            p_ = jnp.exp(scores - smax)
            probs = p_ / jnp.sum(p_, axis=-1, keepdims=True)
            idxh = idx_ref[pl.ds(row_off, HALF), :]
            cols = lax.broadcasted_iota(jnp.int32, (HALF, N_EXPERTS), 1)
            wh = jnp.sum(jnp.where(cols == idxh, probs, 0.0),
                         axis=-1, keepdims=True)
            ph = jnp.zeros((HALF, D_FF), jnp.float32)
            for j in range(E_PER_DEV):
                e = me * E_PER_DEV + j
                yj = jnp.dot(xbh, ew_ref[j].astype(jnp.bfloat16),
                             preferred_element_type=jnp.float32)
                ph = ph + jnp.where(idxh == e, wh, 0.0) * yj
            acc_ref[pl.ds(row_off, HALF), :] = ph.astype(jnp.bfloat16)

        partner_x, bit_x, _ = steps[0]
        my0 = cx * HALF
        peer0 = (1 - cx) * HALF
        compute_partial(peer0)
        rdma_x = pltpu.make_async_remote_copy(
            src_ref=acc_ref.at[pl.ds(peer0, HALF)],
            dst_ref=comm_ref.at[pl.ds(0, HALF)],
            send_sem=send_sems.at[0], recv_sem=recv_sems.at[0],
            device_id=(partner_x,), device_id_type=pl.DeviceIdType.MESH)
        rdma_x.start()
        compute_partial(my0)
        out_ref[:, :] = jnp.dot(x_ref[:, :].astype(jnp.bfloat16),
                                sw_ref[:, :].astype(jnp.bfloat16),
                                preferred_element_type=jnp.float32)
        rdma_x.wait()
        acc_ref[pl.ds(my0, HALF)] = (
            acc_ref[pl.ds(my0, HALF)] + comm_ref[pl.ds(0, HALF)])

        off = my0
        stage = HALF
        for k, (partner, bit, sz) in enumerate(steps[1:], start=1):
            my_off = off + bit * sz
            peer_off = off + (1 - bit) * sz
            rdma = pltpu.make_async_remote_copy(
                src_ref=acc_ref.at[pl.ds(peer_off, sz)],
                dst_ref=comm_ref.at[pl.ds(stage, sz)],
                send_sem=send_sems.at[k], recv_sem=recv_sems.at[k],
                device_id=(partner,), device_id_type=pl.DeviceIdType.MESH)
            rdma.start()
            rdma.wait()
            acc_ref[pl.ds(my_off, sz)] = (
                acc_ref[pl.ds(my_off, sz)] + comm_ref[pl.ds(stage, sz)])
            off = my_off
            stage += sz

        for k in range(len(steps)):
            partner, bit, sz = steps[len(steps) - 1 - k]
            rdma = pltpu.make_async_remote_copy(
                src_ref=acc_ref.at[pl.ds(off, sz)],
                dst_ref=acc_ref.at[pl.ds(off, sz)],
                send_sem=send_sems.at[len(steps) + k],
                recv_sem=recv_sems.at[len(steps) + k],
                device_id=(partner,), device_id_type=pl.DeviceIdType.MESH)
            rdma.start()
            if sz == HALF:
                out_ref[pl.ds(off, HALF), :] = (
                    out_ref[pl.ds(off, HALF), :]
                    + acc_ref[pl.ds(off, HALF), :].astype(jnp.float32))
            rdma.wait()
            off = off - bit * sz

        other0 = (1 - cx) * HALF
        out_ref[pl.ds(other0, HALF), :] = (
            out_ref[pl.ds(other0, HALF), :]
            + acc_ref[pl.ds(other0, HALF), :].astype(jnp.float32))

        for p, _, _ in steps:
            pl.semaphore_signal(exit_sem, inc=1, device_id=(p,),
                                device_id_type=pl.DeviceIdType.MESH)
        pl.semaphore_wait(exit_sem, len(steps))

    return pl.pallas_call(
        body,
        out_shape=jax.ShapeDtypeStruct((N_ROWS, D_FF), jnp.float32),
        in_specs=[pl.BlockSpec(memory_space=pltpu.VMEM)] * 5,
        out_specs=pl.BlockSpec(memory_space=pltpu.VMEM),
        scratch_shapes=[
            pltpu.VMEM((N_ROWS, D_FF), jnp.bfloat16),
            pltpu.VMEM((992, D_FF), jnp.bfloat16),
            pltpu.SemaphoreType.DMA((10,)),
            pltpu.SemaphoreType.DMA((10,)),
            pltpu.SemaphoreType.REGULAR,
        ],
        compiler_params=pltpu.CompilerParams(collective_id=0),
    )(x, router_W, route_idx, expert_W, shared_W)
